# Initial kernel scaffold; baseline (speedup 1.0000x reference)
import jax
import jax.numpy as jnp
from jax.experimental import pallas as pl


def kernel(logits):
    raise NotImplementedError("write your pallas kernel here")


# speedup vs baseline: 3.1001x; 3.1001x over previous
import jax
import jax.numpy as jnp
from jax.experimental import pallas as pl

B, S, V = 64, 16, 32768
ROWS = B * S
BLOCK_ROWS = 8


def _onehot_body(x_ref, o_ref):
    x = x_ref[...]
    m = jnp.max(x, axis=1, keepdims=True)
    iota = jax.lax.broadcasted_iota(jnp.int32, x.shape, 1)
    idx = jnp.min(jnp.where(x == m, iota, V), axis=1, keepdims=True)
    o_ref[...] = (iota == idx).astype(jnp.float32)


def kernel(logits):
    x = logits.reshape(ROWS, V)
    out = pl.pallas_call(
        _onehot_body,
        grid=(ROWS // BLOCK_ROWS,),
        in_specs=[pl.BlockSpec((BLOCK_ROWS, V), lambda i: (i, 0))],
        out_specs=pl.BlockSpec((BLOCK_ROWS, V), lambda i: (i, 0)),
        out_shape=jax.ShapeDtypeStruct((ROWS, V), jnp.float32),
    )(x)
    return out.reshape(B, S, V)


# speedup vs baseline: 5.5769x; 1.7989x over previous
import jax
import jax.numpy as jnp
from jax.experimental import pallas as pl

B, S, V = 64, 16, 32768
ROWS = B * S
BLOCK_ROWS = 32


def _onehot_body(x_ref, o_ref):
    x = x_ref[...]
    m = jnp.max(x, axis=1, keepdims=True)
    iota = jax.lax.broadcasted_iota(jnp.int32, x.shape, 1)
    idx = jnp.min(jnp.where(x == m, iota, V), axis=1, keepdims=True)
    o_ref[...] = (iota == idx).astype(jnp.float32)


def kernel(logits):
    x = logits.reshape(ROWS, V)
    out = pl.pallas_call(
        _onehot_body,
        grid=(ROWS // BLOCK_ROWS,),
        in_specs=[pl.BlockSpec((BLOCK_ROWS, V), lambda i: (i, 0))],
        out_specs=pl.BlockSpec((BLOCK_ROWS, V), lambda i: (i, 0)),
        out_shape=jax.ShapeDtypeStruct((ROWS, V), jnp.float32),
    )(x)
    return out.reshape(B, S, V)


# speedup vs baseline: 5.8050x; 1.0409x over previous
import jax
import jax.numpy as jnp
from jax.experimental import pallas as pl

B, S, V = 64, 16, 32768
ROWS = B * S
BLOCK_ROWS = 64


def _onehot_body(x_ref, o_ref):
    x = x_ref[...]
    m = jnp.max(x, axis=1, keepdims=True)
    iota = jax.lax.broadcasted_iota(jnp.int32, x.shape, 1)
    idx = jnp.min(jnp.where(x == m, iota, V), axis=1, keepdims=True)
    o_ref[...] = (iota == idx).astype(jnp.float32)


def kernel(logits):
    x = logits.reshape(ROWS, V)
    out = pl.pallas_call(
        _onehot_body,
        grid=(ROWS // BLOCK_ROWS,),
        in_specs=[pl.BlockSpec((BLOCK_ROWS, V), lambda i: (i, 0))],
        out_specs=pl.BlockSpec((BLOCK_ROWS, V), lambda i: (i, 0)),
        out_shape=jax.ShapeDtypeStruct((ROWS, V), jnp.float32),
    )(x)
    return out.reshape(B, S, V)
